# Initial kernel scaffold; baseline (speedup 1.0000x reference)
import jax
import jax.numpy as jnp
from jax.experimental import pallas as pl


def kernel(x, target):
    raise NotImplementedError("write your pallas kernel here")


# speedup vs baseline: 11.3514x; 11.3514x over previous
import functools

import jax
import jax.numpy as jnp
from jax import lax
from jax.experimental import pallas as pl
from jax.experimental.pallas import tpu as pltpu

_SIZE = 16384
_N = 4096
_SMOOTH = 0.1
_CONF = 1.0 - _SMOOTH
_S = _SMOOTH / (_SIZE - 2)
import math as _math
_C0 = (_SIZE - 2) * _S * _math.log(_S) + _CONF * _math.log(_CONF)

_R = 256


def _body(tgt_ref, x_ref, out_ref):
    i = pl.program_id(0)
    t = tgt_ref[0, 0, :]
    x = x_ref[...]
    cols = lax.broadcasted_iota(jnp.int32, (_R, _SIZE), 1)
    tcol = t[:, None]
    td = jnp.where(cols == tcol, jnp.float32(_CONF), jnp.float32(_S))
    td = jnp.where(cols == 0, jnp.float32(0.0), td)
    td = jnp.where(tcol == 0, jnp.float32(0.0), td)
    partial = jnp.sum(td * x)
    nm = jnp.sum(jnp.where(t != 0, jnp.float32(1.0), jnp.float32(0.0)))
    val = jnp.float32(_C0) * nm - partial

    @pl.when(i == 0)
    def _():
        out_ref[0, 0] = jnp.float32(0.0)

    out_ref[0, 0] += val


@jax.jit
def kernel(x, target):
    nr = _N // _R
    tgt3 = target.astype(jnp.int32).reshape(nr, 1, _R)
    out = pl.pallas_call(
        _body,
        grid=(nr,),
        in_specs=[
            pl.BlockSpec((1, 1, _R), lambda i: (i, 0, 0)),
            pl.BlockSpec((_R, _SIZE), lambda i: (i, 0)),
        ],
        out_specs=pl.BlockSpec(
            (1, 1), lambda i: (0, 0), memory_space=pltpu.SMEM
        ),
        out_shape=jax.ShapeDtypeStruct((1, 1), jnp.float32),
    )(tgt3, x)
    return out[0, 0]
